# speedup vs baseline: 1.0556x; 1.0556x over previous
import functools

import jax
import jax.numpy as jnp
from jax.experimental import pallas as pl


def _mm_kernel(x_ref, w_ref, o_ref):
    o_ref[...] = jnp.dot(x_ref[...], w_ref[...],
                         preferred_element_type=jnp.float32)


def _matmul(x, W):
    M, K = x.shape
    _, N = W.shape
    BM = 1264
    return pl.pallas_call(
        _mm_kernel,
        grid=(M // BM,),
        in_specs=[pl.BlockSpec((BM, K), lambda i: (i, 0)),
                  pl.BlockSpec((K, N), lambda i: (0, 0))],
        out_specs=pl.BlockSpec((BM, N), lambda i: (i, 0)),
        out_shape=jax.ShapeDtypeStruct((M, N), jnp.float32),
    )(x, W)


def kernel(x, edge_index, W0, b0, W1, b1, W2, b2):
    n = x.shape[0]
    dt = x.dtype
    loop = jnp.arange(n, dtype=edge_index.dtype)
    src = jnp.concatenate([edge_index[0], loop])
    dst = jnp.concatenate([edge_index[1], loop])
    deg = jax.ops.segment_sum(jnp.ones_like(src, dtype=dt), dst, num_segments=n)
    dinv = jnp.where(deg > 0, 1.0 / jnp.sqrt(deg), 0.0)
    norm = dinv[src] * dinv[dst]

    npad = 10112
    xp = jnp.pad(x, ((0, npad - n), (0, 0)))
    W2p = jnp.pad(W2, ((0, 0), (0, 128 - W2.shape[1])))

    def layer(h, W, b):
        hw = _matmul(h, W)[:n]
        msg = hw[src] * norm[:, None]
        out = jax.ops.segment_sum(msg, dst, num_segments=n)
        return out + b

    h = jax.nn.relu(layer(xp, W0, b0))
    h = jax.nn.relu(layer(jnp.pad(h, ((0, npad - n), (0, 0))), W1, b1))
    out = layer(jnp.pad(h, ((0, npad - n), (0, 0))), W2p,
                jnp.pad(b2, (0, 128 - b2.shape[0])))
    return out[:, :W2.shape[1]]


# speedup vs baseline: 12.1672x; 11.5266x over previous
import functools

import jax
import jax.numpy as jnp
from jax import lax
from jax.experimental import pallas as pl
from jax.experimental.pallas import tpu as pltpu
from jax.experimental.pallas import tpu_sc as plsc

N = 10000
NPAD = 10112
E = 320000
EPT = 10112
NB = 79
BATCH = 128
SHARE = NPAD // 16
BM = 1264
H = 128
WC = 16

_MESH = plsc.VectorSubcoreMesh(core_axis_name="c", subcore_axis_name="s")


def _agg_call(hp, sd):
    @functools.partial(
        pl.kernel,
        mesh=_MESH,
        out_type=jax.ShapeDtypeStruct((2, NPAD, H), jnp.float32),
        scratch_types=[
            pltpu.VMEM((2, BATCH), jnp.int32),
            pltpu.VMEM((2, BATCH), jnp.int32),
            pltpu.VMEM((BATCH, H), jnp.float32),
            pltpu.VMEM((BATCH, H), jnp.float32),
            pltpu.VMEM_SHARED((NPAD, H), jnp.float32),
            pltpu.SemaphoreType.DMA,
            pltpu.SemaphoreType.DMA,
            pltpu.SemaphoreType.DMA,
            pltpu.SemaphoreType.DMA,
        ],
    )
    def agg(hp_hbm, sd_hbm, out_hbm,
            sdb0, sdb1, buf0, buf1, acc_sh, gsem0, gsem1, isem0, isem1):
        c = lax.axis_index("c")
        s = lax.axis_index("s")
        wid = c * 16 + s
        base = s * SHARE
        pltpu.sync_copy(hp_hbm.at[pl.ds(base, SHARE)],
                        acc_sh.at[pl.ds(base, SHARE)])
        plsc.subcore_barrier()

        def idx_start(j, sdb, isem):
            pltpu.async_copy(sd_hbm.at[wid, j], sdb, isem)

        def idx_wait(j, sdb, isem):
            pltpu.make_async_copy(sd_hbm.at[wid, j], sdb, isem).wait()

        def gather_start(sdb, buf, gsem):
            pltpu.async_copy(hp_hbm.at[sdb.at[0]], buf, gsem)

        def gather_wait(sdb, buf, gsem):
            pltpu.make_async_copy(hp_hbm.at[sdb.at[0]], buf, gsem).wait()

        def scat(sdb, buf):
            pltpu.sync_copy(buf, acc_sh.at[sdb.at[1]], add=True)

        pltpu.sync_copy(sd_hbm.at[wid, 0], sdb0)
        gather_start(sdb0, buf0, gsem0)
        idx_start(1, sdb1, isem1)

        def body(i, carry):
            j1 = 2 * i + 1
            j2 = 2 * i + 2
            j3 = 2 * i + 3

            @pl.when(j1 < NB)
            def _():
                idx_wait(j1, sdb1, isem1)
                gather_start(sdb1, buf1, gsem1)

            gather_wait(sdb0, buf0, gsem0)
            scat(sdb0, buf0)

            @pl.when(j2 < NB)
            def _():
                idx_start(j2, sdb0, isem0)

            @pl.when(j1 < NB)
            def _():
                @pl.when(j2 < NB)
                def _():
                    idx_wait(j2, sdb0, isem0)
                    gather_start(sdb0, buf0, gsem0)

                gather_wait(sdb1, buf1, gsem1)
                scat(sdb1, buf1)

                @pl.when(j3 < NB)
                def _():
                    idx_start(j3, sdb1, isem1)

            return carry

        lax.fori_loop(0, (NB + 1) // 2, body, 0)
        plsc.subcore_barrier()
        pltpu.sync_copy(acc_sh.at[pl.ds(base, SHARE)],
                        out_hbm.at[c, pl.ds(base, SHARE)])

    return agg(hp, sd)


def _deg_call(dstg, zer, one):
    @functools.partial(
        pl.kernel,
        mesh=_MESH,
        out_type=jax.ShapeDtypeStruct((2, NPAD, WC), jnp.float32),
        scratch_types=[
            pltpu.VMEM((NB, BATCH), jnp.int32),
            pltpu.VMEM((BATCH, WC), jnp.float32),
            pltpu.VMEM_SHARED((NPAD, WC), jnp.float32),
        ],
    )
    def deg(dst_hbm, zer_hbm, one_hbm, out_hbm, dst_v, ones_v, acc_sh):
        c = lax.axis_index("c")
        s = lax.axis_index("s")
        wid = c * 16 + s
        pltpu.sync_copy(dst_hbm.at[wid], dst_v)
        pltpu.sync_copy(one_hbm, ones_v)
        base = s * SHARE
        pltpu.sync_copy(zer_hbm, acc_sh.at[pl.ds(base, SHARE)])
        plsc.subcore_barrier()

        def body(j, carry):
            pltpu.sync_copy(ones_v, acc_sh.at[dst_v.at[j]], add=True)
            return carry

        lax.fori_loop(0, NB, body, 0)
        plsc.subcore_barrier()
        pltpu.sync_copy(acc_sh.at[pl.ds(base, SHARE)],
                        out_hbm.at[c, pl.ds(base, SHARE)])

    return deg(dstg, zer, one)


def _row_spec(w):
    return pl.BlockSpec((BM, w), lambda i: (i, 0))


def _full_spec(r, c):
    return pl.BlockSpec((r, c), lambda i: (0, 0))


def _k1_body(x_ref, w_ref, d0_ref, d1_ref, hp_ref, dinvb_ref):
    deg = d0_ref[:, :1] + d1_ref[:, :1] + 1.0
    dinv = lax.rsqrt(deg)
    dinvb = jnp.broadcast_to(dinv, (BM, H))
    hp_ref[...] = dinvb * jnp.dot(x_ref[...], w_ref[...],
                                  preferred_element_type=jnp.float32)
    dinvb_ref[...] = dinvb


def _k1(xp, W0, d0, d1):
    return pl.pallas_call(
        _k1_body,
        grid=(NPAD // BM,),
        in_specs=[_row_spec(H), _full_spec(H, H), _row_spec(WC), _row_spec(WC)],
        out_specs=[_row_spec(H), _row_spec(H)],
        out_shape=[jax.ShapeDtypeStruct((NPAD, H), jnp.float32),
                   jax.ShapeDtypeStruct((NPAD, H), jnp.float32)],
    )(xp, W0, d0, d1)


def _k2_body(a0_ref, a1_ref, hp_ref, dinvb_ref, b_ref, w_ref, o_ref):
    pre = (a0_ref[...] + a1_ref[...] - hp_ref[...]) * dinvb_ref[...] + b_ref[...]
    x1 = jnp.maximum(pre, 0.0)
    o_ref[...] = dinvb_ref[...] * jnp.dot(x1, w_ref[...],
                                          preferred_element_type=jnp.float32)


def _k2(a0, a1, hp, dinvb, brow, W):
    return pl.pallas_call(
        _k2_body,
        grid=(NPAD // BM,),
        in_specs=[_row_spec(H), _row_spec(H), _row_spec(H), _row_spec(H),
                  _full_spec(1, H), _full_spec(H, H)],
        out_specs=_row_spec(H),
        out_shape=jax.ShapeDtypeStruct((NPAD, H), jnp.float32),
    )(a0, a1, hp, dinvb, brow, W)


def _k4_body(a0_ref, a1_ref, hp_ref, dinvb_ref, b_ref, o_ref):
    o_ref[...] = ((a0_ref[...] + a1_ref[...] - hp_ref[...]) * dinvb_ref[...]
                  + b_ref[...])


def _k4(a0, a1, hp, dinvb, brow):
    return pl.pallas_call(
        _k4_body,
        grid=(NPAD // BM,),
        in_specs=[_row_spec(H), _row_spec(H), _row_spec(H), _row_spec(H),
                  _full_spec(1, H)],
        out_specs=_row_spec(H),
        out_shape=jax.ShapeDtypeStruct((NPAD, H), jnp.float32),
    )(a0, a1, hp, dinvb, brow)


def kernel(x, edge_index, W0, b0, W1, b1, W2, b2):
    nc = W2.shape[1]
    ei = edge_index.astype(jnp.int32)
    pad_e = 32 * EPT - E
    fill = jnp.full((pad_e,), N, jnp.int32)
    srcg = jnp.concatenate([ei[0], fill]).reshape(32, NB, BATCH)
    dstg = jnp.concatenate([ei[1], fill]).reshape(32, NB, BATCH)
    xp = jnp.pad(x, ((0, NPAD - N), (0, 0)))
    zer = jnp.zeros((SHARE, WC), jnp.float32)
    one = jnp.ones((BATCH, WC), jnp.float32)
    W2p = jnp.pad(W2, ((0, 0), (0, H - nc)))
    b0r = b0.reshape(1, H)
    b1r = b1.reshape(1, H)
    b2r = jnp.pad(b2, (0, H - nc)).reshape(1, H)

    sd = jnp.stack([srcg, dstg], axis=2)
    d = _deg_call(dstg, zer, one)
    hp0, dinvb = _k1(xp, W0, d[0], d[1])
    acc = _agg_call(hp0, sd)
    hp1 = _k2(acc[0], acc[1], hp0, dinvb, b0r, W1)
    acc = _agg_call(hp1, sd)
    hp2 = _k2(acc[0], acc[1], hp1, dinvb, b1r, W2p)
    acc = _agg_call(hp2, sd)
    out = _k4(acc[0], acc[1], hp2, dinvb, b2r)
    return out[:N, :nc]
